# baseline (speedup 1.0000x reference)
import jax
import jax.numpy as jnp
from jax.experimental import pallas as pl
from jax.experimental.pallas import tpu as pltpu
from jax.experimental.pallas import tpu_sc as plsc

_B = 4096
_L = 10
_D = 64
_W = 128


def _sc_gather_body(user_hbm, uidx_hbm, item_hbm, iidx_hbm, uout_hbm, iout_hbm):
    def ubody(i_vmem, o_vmem):
        pltpu.sync_copy(user_hbm.at[i_vmem.at[0]], o_vmem)

    pltpu.emit_pipeline(
        ubody,
        grid=(_B // _W,),
        in_specs=[pl.BlockSpec((1, _W), lambda i: (0, i))],
        out_specs=[pl.BlockSpec((_W, _D), lambda i: (i, 0))],
        core_axis_name=("c", "s"),
        dimension_semantics=(pltpu.PARALLEL,),
    )(uidx_hbm, uout_hbm)

    def ibody(i_vmem, o_vmem):
        pltpu.sync_copy(item_hbm.at[i_vmem.at[0]], o_vmem)

    pltpu.emit_pipeline(
        ibody,
        grid=(_B * _L // _W,),
        in_specs=[pl.BlockSpec((1, _W), lambda i: (0, i))],
        out_specs=[pl.BlockSpec((_W, _D), lambda i: (i, 0))],
        core_axis_name=("c", "s"),
        dimension_semantics=(pltpu.PARALLEL,),
    )(iidx_hbm, iout_hbm)


def _softplus(x):
    return jnp.maximum(x, 0.0) + jnp.log1p(jnp.exp(-jnp.abs(x)))


def _tc_loss_body(u_ref, it_ref, loss_ref, l2_ref):
    u = u_ref[...]
    l2row = jnp.sum(u * u, axis=1)
    z = []
    for l in range(_L):
        itl = it_ref[:, l * _D:(l + 1) * _D]
        z.append(jnp.sum(u * itl, axis=1))
        l2row = l2row + jnp.sum(itl * itl, axis=1)
    z_ai, z_aj = z[0], z[1]
    pos1 = jnp.minimum(jnp.abs(z_ai - z_aj), 0.5)
    m6_sum = None
    m6_max = None
    for k in range(2, _L):
        pn = _softplus(z[k] - z_ai) + _softplus(z[k] - z_aj)
        m6_sum = pn if m6_sum is None else m6_sum + pn
        m6_max = pn if m6_max is None else jnp.maximum(m6_max, pn)
    posdis = _softplus(m6_max - 2.0 * pos1)
    s1 = jnp.sum(posdis) * (1.0 / _B)
    s2 = jnp.sum(m6_sum) * (1.0 / _B)
    l2reg = 0.01 * jnp.sum(l2row) * (1.0 / _B)
    loss_ref[0, 0] = s1 + s2 + l2reg
    l2_ref[0, 0] = l2reg


def kernel(one_batch, embed_user, embed_item):
    uidx = one_batch[:, 0].reshape(1, _B)
    iidx = one_batch[:, 1:].reshape(1, _B * _L)

    mesh = plsc.VectorSubcoreMesh(core_axis_name="c", subcore_axis_name="s")
    gather = pl.kernel(
        _sc_gather_body,
        out_type=(
            jax.ShapeDtypeStruct((_B, _D), jnp.float32),
            jax.ShapeDtypeStruct((_B * _L, _D), jnp.float32),
        ),
        mesh=mesh,
        compiler_params=pltpu.CompilerParams(use_tc_tiling_on_sc=False),
    )
    u, it = gather(embed_user, uidx, embed_item, iidx)

    loss, l2 = pl.pallas_call(
        _tc_loss_body,
        out_shape=(
            jax.ShapeDtypeStruct((1, 1), jnp.float32),
            jax.ShapeDtypeStruct((1, 1), jnp.float32),
        ),
        out_specs=(
            pl.BlockSpec(memory_space=pltpu.SMEM),
            pl.BlockSpec(memory_space=pltpu.SMEM),
        ),
    )(u, it.reshape(_B, _L * _D))
    return (loss[0, 0], l2[0, 0])


# speedup vs baseline: 1.0027x; 1.0027x over previous
import jax
import jax.numpy as jnp
from jax import lax
from jax.experimental import pallas as pl
from jax.experimental.pallas import tpu as pltpu
from jax.experimental.pallas import tpu_sc as plsc

_B = 4096
_L = 10
_D = 64
_NC = 2
_NS = 16
_NW = _NC * _NS
_WB = _B // _NW
_LANES = 16


def _sc_gather_body(ob_hbm, user_hbm, item_hbm, uout_hbm, iout_hbm,
                    ob_v, uidx_v, iidx_v, urows_v, irows_v, sem):
    wid = lax.axis_index("s") * _NC + lax.axis_index("c")
    base = wid * _WB

    pltpu.sync_copy(ob_hbm.at[pl.ds(base, _WB), :], ob_v)

    lane = lax.iota(jnp.int32, _LANES)
    for g in range(_WB // _LANES):
        rows = lane + (g * _LANES)
        uidx_v[pl.ds(g * _LANES, _LANES)] = plsc.load_gather(
            ob_v, [rows, jnp.zeros((_LANES,), jnp.int32)])
        for l in range(_L):
            iidx_v[l, pl.ds(g * _LANES, _LANES)] = plsc.load_gather(
                ob_v, [rows, jnp.full((_LANES,), l + 1, jnp.int32)])

    copies = [pltpu.async_copy(user_hbm.at[uidx_v], urows_v, sem)]
    for l in range(_L):
        copies.append(
            pltpu.async_copy(item_hbm.at[iidx_v.at[l]], irows_v.at[l], sem))
    for c in copies:
        c.wait()

    pltpu.sync_copy(urows_v, uout_hbm.at[pl.ds(base, _WB)])
    for l in range(_L):
        pltpu.sync_copy(irows_v.at[l], iout_hbm.at[l, pl.ds(base, _WB), :])


def _softplus(x):
    return jnp.maximum(x, 0.0) + jnp.log1p(jnp.exp(-jnp.abs(x)))


def _tc_loss_body(u_ref, it_ref, loss_ref, l2_ref):
    u = u_ref[...]
    l2row = jnp.sum(u * u, axis=1)
    z = []
    for l in range(_L):
        itl = it_ref[l]
        z.append(jnp.sum(u * itl, axis=1))
        l2row = l2row + jnp.sum(itl * itl, axis=1)
    z_ai, z_aj = z[0], z[1]
    pos1 = jnp.minimum(jnp.abs(z_ai - z_aj), 0.5)
    m6_sum = None
    m6_max = None
    for k in range(2, _L):
        pn = _softplus(z[k] - z_ai) + _softplus(z[k] - z_aj)
        m6_sum = pn if m6_sum is None else m6_sum + pn
        m6_max = pn if m6_max is None else jnp.maximum(m6_max, pn)
    posdis = _softplus(m6_max - 2.0 * pos1)
    s1 = jnp.sum(posdis) * (1.0 / _B)
    s2 = jnp.sum(m6_sum) * (1.0 / _B)
    l2reg = 0.01 * jnp.sum(l2row) * (1.0 / _B)
    loss_ref[0, 0] = s1 + s2 + l2reg
    l2_ref[0, 0] = l2reg


def kernel(one_batch, embed_user, embed_item):
    mesh = plsc.VectorSubcoreMesh(core_axis_name="c", subcore_axis_name="s")
    gather = pl.kernel(
        _sc_gather_body,
        out_type=(
            jax.ShapeDtypeStruct((_B, _D), jnp.float32),
            jax.ShapeDtypeStruct((_L, _B, _D), jnp.float32),
        ),
        mesh=mesh,
        scratch_types=[
            pltpu.VMEM((_WB, 11), jnp.int32),
            pltpu.VMEM((_WB,), jnp.int32),
            pltpu.VMEM((_L, _WB), jnp.int32),
            pltpu.VMEM((_WB, _D), jnp.float32),
            pltpu.VMEM((_L, _WB, _D), jnp.float32),
            pltpu.SemaphoreType.DMA,
        ],
        compiler_params=pltpu.CompilerParams(
            use_tc_tiling_on_sc=False, needs_layout_passes=False),
    )
    u, it = gather(one_batch, embed_user, embed_item)

    loss, l2 = pl.pallas_call(
        _tc_loss_body,
        out_shape=(
            jax.ShapeDtypeStruct((1, 1), jnp.float32),
            jax.ShapeDtypeStruct((1, 1), jnp.float32),
        ),
        out_specs=(
            pl.BlockSpec(memory_space=pltpu.SMEM),
            pl.BlockSpec(memory_space=pltpu.SMEM),
        ),
    )(u, it)
    return (loss[0, 0], l2[0, 0])
